# baseline (speedup 1.0000x reference)
import functools

import jax
import jax.numpy as jnp
from jax import lax
from jax.experimental import pallas as pl
from jax.experimental.pallas import tpu as pltpu
from jax.experimental.pallas import tpu_sc as plsc

N_NODES = 10000
N_EDGES = 320000
D_FEAT = 128

SC_CORES = 2
SC_SUBCORES = 16
SC_LANES = 16
N_WORKERS = SC_CORES * SC_SUBCORES
MAIN_PER_WORKER = (N_EDGES // N_WORKERS) // 128 * 128
TAIL_BASE = N_WORKERS * MAIN_PER_WORKER
TAIL_BLOCKS = (N_EDGES - TAIL_BASE) // 128
BUF = MAIN_PER_WORKER + 128
N_PAD = 10240


def _node_table_body(h_ref, wt_ref, b_ref, out_ref):
    res = lax.dot_general(
        wt_ref[...], h_ref[...],
        dimension_numbers=(((1,), (1,)), ((), ())),
        preferred_element_type=jnp.float32,
    )
    row = lax.broadcasted_iota(jnp.int32, res.shape, 0)
    out_ref[...] = res + jnp.where(row == 0, b_ref[0], 0.0)


def _node_table(h, wt, b):
    blk = 5120
    return pl.pallas_call(
        _node_table_body,
        grid=(N_PAD // blk,),
        out_shape=jax.ShapeDtypeStruct((2, N_PAD), jnp.float32),
        in_specs=[
            pl.BlockSpec((blk, D_FEAT), lambda i: (i, 0)),
            pl.BlockSpec((2, D_FEAT), lambda i: (0, 0)),
            pl.BlockSpec(memory_space=pltpu.SMEM),
        ],
        out_specs=pl.BlockSpec((2, blk), lambda i: (0, i)),
    )(h, wt, b)


def _edge_scores(pq, edge_index):
    mesh = plsc.VectorSubcoreMesh(core_axis_name="c", subcore_axis_name="s")

    @functools.partial(
        pl.kernel,
        mesh=mesh,
        out_type=jax.ShapeDtypeStruct((N_EDGES,), jnp.float32),
        compiler_params=pltpu.CompilerParams(needs_layout_passes=False),
        scratch_types=[
            pltpu.VMEM((N_PAD,), jnp.float32),
            pltpu.VMEM((N_PAD,), jnp.float32),
            pltpu.VMEM((2, BUF), jnp.int32),
            pltpu.VMEM((BUF,), jnp.float32),
            pltpu.VMEM_SHARED((2, N_PAD), jnp.float32),
            pltpu.SemaphoreType.DMA,
            pltpu.SemaphoreType.DMA,
            pltpu.SemaphoreType.DMA,
            pltpu.SemaphoreType.DMA,
        ],
    )
    def sc_kernel(pq_hbm, ei_hbm, out_hbm, p_v, q_v, sd_v, o_v, pq_sh,
                  s0, s1, s2, s3):
        sid = lax.axis_index("s")
        wid = sid * SC_CORES + lax.axis_index("c")
        base = wid * MAIN_PER_WORKER
        tail = TAIL_BASE + wid * 128
        has_tail = wid < TAIL_BLOCKS
        c1 = pltpu.async_copy(
            ei_hbm.at[:, pl.ds(base, MAIN_PER_WORKER)],
            sd_v.at[:, pl.ds(0, MAIN_PER_WORKER)], s1)

        @pl.when(has_tail)
        def _():
            pltpu.async_copy(
                ei_hbm.at[:, pl.ds(tail, 128)],
                sd_v.at[:, pl.ds(MAIN_PER_WORKER, 128)], s2).wait()

        half = (N_PAD // 128 // 2) * 128

        @pl.when(sid == 0)
        def _():
            pltpu.sync_copy(pq_hbm.at[:, pl.ds(0, half)],
                            pq_sh.at[:, pl.ds(0, half)])

        @pl.when(sid == 1)
        def _():
            pltpu.sync_copy(pq_hbm.at[:, pl.ds(half, N_PAD - half)],
                            pq_sh.at[:, pl.ds(half, N_PAD - half)])

        plsc.subcore_barrier()
        c0 = pltpu.async_copy(pq_sh.at[0], p_v, s0)
        c3 = pltpu.async_copy(pq_sh.at[1], q_v, s3)
        c0.wait()
        c3.wait()
        c1.wait()

        def score_block(i):
            sl = pl.ds(i, SC_LANES)
            pv = plsc.load_gather(p_v, [sd_v[0, sl]])
            qv = plsc.load_gather(q_v, [sd_v[1, sl]])
            o_v[sl] = pv + qv

        chunk = MAIN_PER_WORKER // 4
        for k in range(4):
            plsc.parallel_loop(k * chunk, (k + 1) * chunk, step=SC_LANES,
                               unroll=8)(score_block)
            pltpu.async_copy(
                o_v.at[pl.ds(k * chunk, chunk)],
                out_hbm.at[pl.ds(base + k * chunk, chunk)], s1)
        for _ in range(4):
            pltpu.make_async_copy(
                o_v.at[pl.ds(0, chunk)],
                out_hbm.at[pl.ds(base, chunk)], s1).wait()

        @pl.when(has_tail)
        def _():
            plsc.parallel_loop(MAIN_PER_WORKER, BUF, step=SC_LANES, unroll=8)(score_block)
            pltpu.sync_copy(
                o_v.at[pl.ds(MAIN_PER_WORKER, 128)],
                out_hbm.at[pl.ds(tail, 128)])

    return sc_kernel(pq, edge_index).reshape(N_EDGES, 1)


def kernel(h, edge_index, W, b):
    wt = W.reshape(2, D_FEAT)
    pq = _node_table(h, wt, b)
    return _edge_scores(pq, edge_index.astype(jnp.int32))


# speedup vs baseline: 1.0007x; 1.0007x over previous
import functools

import jax
import jax.numpy as jnp
from jax import lax
from jax.experimental import pallas as pl
from jax.experimental.pallas import tpu as pltpu
from jax.experimental.pallas import tpu_sc as plsc

N_NODES = 10000
N_EDGES = 320000
D_FEAT = 128

SC_CORES = 2
SC_SUBCORES = 16
SC_LANES = 16
N_WORKERS = SC_CORES * SC_SUBCORES
MAIN_PER_WORKER = (N_EDGES // N_WORKERS) // 128 * 128
TAIL_BASE = N_WORKERS * MAIN_PER_WORKER
TAIL_BLOCKS = (N_EDGES - TAIL_BASE) // 128
BUF = MAIN_PER_WORKER + 128
N_PAD = 10240


def _node_table_body(h_ref, wt_ref, b_ref, out_ref):
    res = lax.dot_general(
        wt_ref[...], h_ref[...],
        dimension_numbers=(((1,), (1,)), ((), ())),
        preferred_element_type=jnp.float32,
    )
    row = lax.broadcasted_iota(jnp.int32, res.shape, 0)
    out_ref[...] = res + jnp.where(row == 0, b_ref[0], 0.0)


def _node_table(h, wt, b):
    blk = 5120
    return pl.pallas_call(
        _node_table_body,
        grid=(N_PAD // blk,),
        out_shape=jax.ShapeDtypeStruct((2, N_PAD), jnp.float32),
        in_specs=[
            pl.BlockSpec((blk, D_FEAT), lambda i: (i, 0)),
            pl.BlockSpec((2, D_FEAT), lambda i: (0, 0)),
            pl.BlockSpec(memory_space=pltpu.SMEM),
        ],
        out_specs=pl.BlockSpec((2, blk), lambda i: (0, i)),
    )(h, wt, b)


def _edge_scores(pq, edge_index):
    mesh = plsc.VectorSubcoreMesh(core_axis_name="c", subcore_axis_name="s")

    @functools.partial(
        pl.kernel,
        mesh=mesh,
        out_type=jax.ShapeDtypeStruct((N_EDGES,), jnp.float32),
        compiler_params=pltpu.CompilerParams(needs_layout_passes=False),
        scratch_types=[
            pltpu.VMEM((N_PAD,), jnp.float32),
            pltpu.VMEM((N_PAD,), jnp.float32),
            pltpu.VMEM((2, BUF), jnp.int32),
            pltpu.VMEM((BUF,), jnp.float32),
            pltpu.VMEM_SHARED((2, N_PAD), jnp.float32),
            pltpu.SemaphoreType.DMA,
            pltpu.SemaphoreType.DMA,
            pltpu.SemaphoreType.DMA,
            pltpu.SemaphoreType.DMA,
        ],
    )
    def sc_kernel(pq_hbm, ei_hbm, out_hbm, p_v, q_v, sd_v, o_v, pq_sh,
                  s0, s1, s2, s3):
        sid = lax.axis_index("s")
        wid = sid * SC_CORES + lax.axis_index("c")
        base = wid * MAIN_PER_WORKER
        tail = TAIL_BASE + wid * 128
        has_tail = wid < TAIL_BLOCKS
        c1 = pltpu.async_copy(
            ei_hbm.at[:, pl.ds(base, MAIN_PER_WORKER)],
            sd_v.at[:, pl.ds(0, MAIN_PER_WORKER)], s1)

        @pl.when(has_tail)
        def _():
            pltpu.async_copy(
                ei_hbm.at[:, pl.ds(tail, 128)],
                sd_v.at[:, pl.ds(MAIN_PER_WORKER, 128)], s2).wait()

        half = (N_PAD // 128 // 2) * 128

        @pl.when(sid == 0)
        def _():
            pltpu.sync_copy(pq_hbm.at[:, pl.ds(0, half)],
                            pq_sh.at[:, pl.ds(0, half)])

        @pl.when(sid == 1)
        def _():
            pltpu.sync_copy(pq_hbm.at[:, pl.ds(half, N_PAD - half)],
                            pq_sh.at[:, pl.ds(half, N_PAD - half)])

        plsc.subcore_barrier()
        c0 = pltpu.async_copy(pq_sh.at[0], p_v, s0)
        c3 = pltpu.async_copy(pq_sh.at[1], q_v, s3)
        c0.wait()
        c3.wait()
        c1.wait()

        def score_block(i):
            sl = pl.ds(i, SC_LANES)
            pv = plsc.load_gather(p_v, [sd_v[0, sl]])
            qv = plsc.load_gather(q_v, [sd_v[1, sl]])
            o_v[sl] = pv + qv

        plsc.parallel_loop(0, MAIN_PER_WORKER, step=SC_LANES, unroll=4)(score_block)
        pltpu.sync_copy(
            o_v.at[pl.ds(0, MAIN_PER_WORKER)],
            out_hbm.at[pl.ds(base, MAIN_PER_WORKER)])

        @pl.when(has_tail)
        def _():
            plsc.parallel_loop(MAIN_PER_WORKER, BUF, step=SC_LANES, unroll=4)(score_block)
            pltpu.sync_copy(
                o_v.at[pl.ds(MAIN_PER_WORKER, 128)],
                out_hbm.at[pl.ds(tail, 128)])

    return sc_kernel(pq, edge_index).reshape(N_EDGES, 1)


def kernel(h, edge_index, W, b):
    wt = W.reshape(2, D_FEAT)
    pq = _node_table(h, wt, b)
    return _edge_scores(pq, edge_index.astype(jnp.int32))
